# baseline (speedup 1.0000x reference)
import jax
import jax.numpy as jnp
from jax import lax
from jax.experimental import pallas as pl
from jax.experimental.pallas import tpu as pltpu
from jax.experimental.pallas import tpu_sc as plsc

N = 10000
D = 128
E = 320000

NC = 2
NS = 16
B = 1024
NB = 320
EP = NB * B
BPW = NB // NS
NPH = 5120
NP = NC * NPH
DUMMY = NPH
ACCR = NPH + 8
RPS = NPH // NS


def _sc_body(x_hbm, row_hbm, col_hbm, valb_hbm, bnds_hbm, z2_hbm, z1_hbm,
             sum_hbm, deg_hbm,
             row_v, loc_v, col_v, valb_v, gx, ones1, bnd_v, db, acc, deg1,
             sem, isem, ssem):
    c = 1 - lax.axis_index("c")
    s = lax.axis_index("s")
    lo = c * NPH
    hi = lo + NPH

    def _fill1(r, carry):
        ones1[pl.ds(r * 16, 16)] = jnp.ones((16,), jnp.float32)
        return carry
    lax.fori_loop(0, 128 // 16, _fill1, 0)

    def _zrow(r, carry):
        for j in range(D // 16):
            gx[r, pl.ds(j * 16, 16)] = jnp.zeros((16,), jnp.float32)
        return carry
    lax.fori_loop(0, RPS + 8, _zrow, 0)

    def _zdg(r, carry):
        db[pl.ds(r * 16, 16)] = jnp.zeros((16,), jnp.float32)
        return carry
    lax.fori_loop(0, RPS // 16, _zdg, 0)

    pltpu.sync_copy(gx.at[pl.ds(0, RPS)], acc.at[pl.ds(s * RPS, RPS)])

    @pl.when(s == 0)
    def _():
        pltpu.sync_copy(gx.at[pl.ds(0, 8)], acc.at[pl.ds(NPH, 8)])
        pltpu.sync_copy(db.at[pl.ds(0, 8)], deg1.at[pl.ds(NPH, 8)])
    pltpu.sync_copy(db, deg1.at[pl.ds(s * RPS, RPS)])

    pltpu.sync_copy(bnds_hbm.at[c], bnd_v)
    blo = jnp.max(bnd_v[0, pl.ds(0, 16)])
    bhi = jnp.max(bnd_v[1, pl.ds(0, 16)])
    plsc.subcore_barrier()

    ilo = jnp.maximum((blo - s + NS - 1) // NS, 0)
    ihi = jnp.minimum((bhi - s + NS - 1) // NS, BPW)

    def _block(i, carry):
        bid = s + i * NS
        e0 = bid * B
        rb = pl.multiple_of(e0 // 128, 8)
        ips = [pltpu.async_copy(row_hbm.at[pl.ds(rb, B // 128)], row_v, isem),
               pltpu.async_copy(col_hbm.at[pl.ds(rb, B // 128)], col_v, isem),
               pltpu.async_copy(valb_hbm.at[pl.ds(e0 * 16, B * 16)],
                                valb_v, isem)]
        for cp in ips:
            cp.wait()

        def _remap(q, carry2):
            for j in range(8):
                rv = row_v[q, pl.ds(j * 16, 16)]
                lr = rv - lo
                m = jnp.logical_and(rv >= lo, rv < hi)
                loc_v[q, pl.ds(j * 16, 16)] = jnp.where(
                    m, lr, jnp.full((16,), DUMMY, jnp.int32))
            return carry2
        lax.fori_loop(0, B // 128, _remap, 0)

        for h in range(2):
            cps = [pltpu.async_copy(x_hbm.at[col_v.at[h * 4 + k]],
                                    gx.at[pl.ds(k * 128, 128)], sem)
                   for k in range(4)]
            for cp in cps:
                cp.wait()

            def _scale(e, carry2):
                vb = valb_v[pl.ds((h * 512 + e) * 16, 16)]
                for j in range(D // 16):
                    g = gx[e, pl.ds(j * 16, 16)]
                    gx[e, pl.ds(j * 16, 16)] = g * vb
                return carry2
            lax.fori_loop(0, 512, _scale, 0)

            sps = []
            for k in range(4):
                sps.append(pltpu.async_copy(
                    gx.at[pl.ds(k * 128, 128)],
                    acc.at[loc_v.at[h * 4 + k]], ssem, add=True))
                sps.append(pltpu.async_copy(
                    ones1, deg1.at[loc_v.at[h * 4 + k]], ssem, add=True))
            for cp in sps:
                cp.wait()
        return carry
    lax.fori_loop(ilo, ihi, _block, 0)

    plsc.subcore_barrier()

    r0 = s * RPS
    pltpu.sync_copy(acc.at[pl.ds(r0, RPS)], gx.at[pl.ds(0, RPS)])
    pltpu.sync_copy(gx.at[pl.ds(0, RPS)],
                    sum_hbm.at[pl.ds(c * NPH + r0, RPS)])
    pltpu.sync_copy(deg1.at[pl.ds(r0, RPS)], db)
    pltpu.sync_copy(db, deg_hbm.at[pl.ds(c * NPH + r0, RPS)])


@jax.jit
def _sc_propagate(x, row2, col2, valb, bnds, z2, z1):
    mesh = plsc.VectorSubcoreMesh(core_axis_name="c", subcore_axis_name="s",
                                  num_cores=NC, num_subcores=NS)
    return pl.kernel(
        _sc_body,
        out_type=(jax.ShapeDtypeStruct((NP, D), jnp.float32),
                  jax.ShapeDtypeStruct((NP,), jnp.float32)),
        mesh=mesh,
        scratch_types=(
            pltpu.VMEM((B // 128, 128), jnp.int32),
            pltpu.VMEM((B // 128, 128), jnp.int32),
            pltpu.VMEM((B // 128, 128), jnp.int32),
            pltpu.VMEM((B * 16,), jnp.float32),
            pltpu.VMEM((512, D), jnp.float32),
            pltpu.VMEM((128,), jnp.float32),
            pltpu.VMEM((2, 16), jnp.int32),
            pltpu.VMEM((RPS,), jnp.float32),
            pltpu.VMEM_SHARED((ACCR, D), jnp.float32),
            pltpu.VMEM_SHARED((ACCR,), jnp.float32),
            pltpu.SemaphoreType.DMA,
            pltpu.SemaphoreType.DMA,
            pltpu.SemaphoreType.DMA,
        ),
        compiler_params=pltpu.CompilerParams(use_tc_tiling_on_sc=False,
                                             needs_layout_passes=False),
    )(x, row2, col2, valb, bnds, z2, z1)


def _fin_body(p_ref, d_ref, o_ref):
    dg = jnp.maximum(d_ref[...], 1.0)
    o_ref[...] = p_ref[...] / dg


@jax.jit
def _finalize(sum_, deg2d):
    R = 1024
    return pl.pallas_call(
        _fin_body,
        grid=(NP // R,),
        in_specs=[
            pl.BlockSpec((R, D), lambda i: (i, 0)),
            pl.BlockSpec((R, 1), lambda i: (i, 0)),
        ],
        out_specs=pl.BlockSpec((R, D), lambda i: (i, 0)),
        out_shape=jax.ShapeDtypeStruct((NP, D), jnp.float32),
    )(sum_, deg2d)


def kernel(x, edge_index, edge_values):
    row = edge_index[0].astype(jnp.int32)
    col = edge_index[1].astype(jnp.int32)
    val = edge_values.astype(jnp.float32)
    pad = EP - E
    row = jnp.concatenate([row, jnp.full((pad,), NP - 8, jnp.int32)])
    col = jnp.concatenate([col, jnp.zeros((pad,), jnp.int32)])
    val = jnp.concatenate([val, jnp.zeros((pad,), jnp.float32)])
    row2 = row.reshape(EP // 128, 128)
    col2 = col.reshape(EP // 128, 128)
    valb = jnp.broadcast_to(val[:, None], (EP, 16)).reshape(EP * 16)
    rowB = row.reshape(NB, B)
    n0 = jnp.searchsorted(rowB[:, 0], NPH).astype(jnp.int32)
    n1 = jnp.searchsorted(rowB[:, B - 1], NPH).astype(jnp.int32)
    bnds = jnp.stack([
        jnp.stack([jnp.zeros((16,), jnp.int32),
                   jnp.broadcast_to(n0, (16,))]),
        jnp.stack([jnp.broadcast_to(n1, (16,)),
                   jnp.full((16,), NB, jnp.int32)]),
    ]).astype(jnp.int32)
    z2 = jnp.zeros((RPS, D), jnp.float32)
    z1 = jnp.zeros((RPS,), jnp.float32)
    sum_, deg = _sc_propagate(x, row2, col2, valb, bnds, z2, z1)
    out = _finalize(sum_, deg.reshape(NP, 1))
    return out[:N]


# speedup vs baseline: 1.4040x; 1.4040x over previous
import jax
import jax.numpy as jnp
from jax import lax
from jax.experimental import pallas as pl
from jax.experimental.pallas import tpu as pltpu
from jax.experimental.pallas import tpu_sc as plsc

N = 10000
D = 128
E = 320000

NC = 2
NS = 16
B = 1024
NB = 320
EP = NB * B
BPW = NB // NS
NPH = 5120
NP = NC * NPH
DUMMY = NPH
ACCR = NPH + 8
RPS = NPH // NS


def _sc_body(x_hbm, row_hbm, col_hbm, valb_hbm, bnds_hbm, z2_hbm, z1_hbm,
             sum_hbm, deg_hbm,
             row_v, loc_v, col_v, valb_v, gx, ones1, bnd_v, db, acc, deg1,
             sem, isem, ssem):
    c = lax.axis_index("c")
    s = lax.axis_index("s")
    lo = c * NPH
    hi = lo + NPH

    def _fill1(r, carry):
        ones1[pl.ds(r * 16, 16)] = jnp.ones((16,), jnp.float32)
        return carry
    lax.fori_loop(0, 128 // 16, _fill1, 0)

    def _zrow(r, carry):
        for j in range(D // 16):
            gx[r, pl.ds(j * 16, 16)] = jnp.zeros((16,), jnp.float32)
        return carry
    lax.fori_loop(0, RPS + 8, _zrow, 0)

    def _zdg(r, carry):
        db[pl.ds(r * 16, 16)] = jnp.zeros((16,), jnp.float32)
        return carry
    lax.fori_loop(0, RPS // 16, _zdg, 0)

    pltpu.sync_copy(gx.at[pl.ds(0, RPS)], acc.at[pl.ds(s * RPS, RPS)])

    @pl.when(s == 0)
    def _():
        pltpu.sync_copy(gx.at[pl.ds(0, 8)], acc.at[pl.ds(NPH, 8)])
        pltpu.sync_copy(db.at[pl.ds(0, 8)], deg1.at[pl.ds(NPH, 8)])
    pltpu.sync_copy(db, deg1.at[pl.ds(s * RPS, RPS)])

    pltpu.sync_copy(bnds_hbm.at[c], bnd_v)
    blo = jnp.max(bnd_v[0, pl.ds(0, 16)])
    bhi = jnp.max(bnd_v[1, pl.ds(0, 16)])
    plsc.subcore_barrier()

    ilo = jnp.maximum((blo - s + NS - 1) // NS, 0)
    ihi = jnp.minimum((bhi - s + NS - 1) // NS, BPW)

    def _block(i, carry):
        bid = s + i * NS
        e0 = bid * B
        rb = pl.multiple_of(e0 // 128, 8)
        ips = [pltpu.async_copy(row_hbm.at[pl.ds(rb, B // 128)], row_v, isem),
               pltpu.async_copy(col_hbm.at[pl.ds(rb, B // 128)], col_v, isem),
               pltpu.async_copy(valb_hbm.at[pl.ds(e0, B)], valb_v, isem)]
        for cp in ips:
            cp.wait()

        def _remap(q, carry2):
            for j in range(8):
                rv = row_v[q, pl.ds(j * 16, 16)]
                lr = rv - lo
                m = jnp.logical_and(rv >= lo, rv < hi)
                loc_v[q, pl.ds(j * 16, 16)] = jnp.where(
                    m, lr, jnp.full((16,), DUMMY, jnp.int32))
            return carry2
        lax.fori_loop(0, B // 128, _remap, 0)

        for h in range(2):
            cps = [pltpu.async_copy(x_hbm.at[col_v.at[h * 4 + k]],
                                    gx.at[pl.ds(k * 128, 128)], sem)
                   for k in range(4)]
            for cp in cps:
                cp.wait()

            def _scale(e, carry2):
                vb = valb_v[pl.ds((h * 512 + e) * 16, 16)]
                for j in range(D // 16):
                    g = gx[e, pl.ds(j * 16, 16)]
                    gx[e, pl.ds(j * 16, 16)] = g * vb
                return carry2
            lax.fori_loop(0, 512, _scale, 0)

            sps = []
            for k in range(4):
                sps.append(pltpu.async_copy(
                    gx.at[pl.ds(k * 128, 128)],
                    acc.at[loc_v.at[h * 4 + k]], ssem, add=True))
                sps.append(pltpu.async_copy(
                    ones1, deg1.at[loc_v.at[h * 4 + k]], ssem, add=True))
            for cp in sps:
                cp.wait()
        return carry
    lax.fori_loop(ilo, ihi, _block, 0)

    plsc.subcore_barrier()

    r0 = s * RPS
    pltpu.sync_copy(acc.at[pl.ds(r0, RPS)], gx.at[pl.ds(0, RPS)])
    pltpu.sync_copy(gx.at[pl.ds(0, RPS)],
                    sum_hbm.at[pl.ds(c * NPH + r0, RPS)])
    pltpu.sync_copy(deg1.at[pl.ds(r0, RPS)], db)
    pltpu.sync_copy(db, deg_hbm.at[pl.ds(c * NPH + r0, RPS)])


@jax.jit
def _sc_propagate(x, row2, col2, valb, bnds, z2, z1):
    mesh = plsc.VectorSubcoreMesh(core_axis_name="c", subcore_axis_name="s",
                                  num_cores=NC, num_subcores=NS)
    return pl.kernel(
        _sc_body,
        out_type=(jax.ShapeDtypeStruct((NP, D), jnp.float32),
                  jax.ShapeDtypeStruct((NP,), jnp.float32)),
        mesh=mesh,
        scratch_types=(
            pltpu.VMEM((B // 128, 128), jnp.int32),
            pltpu.VMEM((B // 128, 128), jnp.int32),
            pltpu.VMEM((B // 128, 128), jnp.int32),
            pltpu.VMEM((B,), jnp.float32),
            pltpu.VMEM((512, D), jnp.float32),
            pltpu.VMEM((128,), jnp.float32),
            pltpu.VMEM((2, 16), jnp.int32),
            pltpu.VMEM((RPS,), jnp.float32),
            pltpu.VMEM_SHARED((ACCR, D), jnp.float32),
            pltpu.VMEM_SHARED((ACCR,), jnp.float32),
            pltpu.SemaphoreType.DMA,
            pltpu.SemaphoreType.DMA,
            pltpu.SemaphoreType.DMA,
        ),
        compiler_params=pltpu.CompilerParams(use_tc_tiling_on_sc=False,
                                             needs_layout_passes=False),
    )(x, row2, col2, valb, bnds, z2, z1)


def _fin_body(p_ref, d_ref, o_ref):
    dg = jnp.maximum(d_ref[...], 1.0)
    o_ref[...] = p_ref[...] / dg


@jax.jit
def _finalize(sum_, deg2d):
    R = 1024
    return pl.pallas_call(
        _fin_body,
        grid=(NP // R,),
        in_specs=[
            pl.BlockSpec((R, D), lambda i: (i, 0)),
            pl.BlockSpec((R, 1), lambda i: (i, 0)),
        ],
        out_specs=pl.BlockSpec((R, D), lambda i: (i, 0)),
        out_shape=jax.ShapeDtypeStruct((NP, D), jnp.float32),
    )(sum_, deg2d)


def kernel(x, edge_index, edge_values):
    row = edge_index[0].astype(jnp.int32)
    col = edge_index[1].astype(jnp.int32)
    val = edge_values.astype(jnp.float32)
    pad = EP - E
    row = jnp.concatenate([row, jnp.full((pad,), NP - 8, jnp.int32)])
    col = jnp.concatenate([col, jnp.zeros((pad,), jnp.int32)])
    val = jnp.concatenate([val, jnp.zeros((pad,), jnp.float32)])
    row2 = row.reshape(EP // 128, 128)
    col2 = col.reshape(EP // 128, 128)
    valb = val
    rowB = row.reshape(NB, B)
    n0 = jnp.searchsorted(rowB[:, 0], NPH).astype(jnp.int32)
    n1 = jnp.searchsorted(rowB[:, B - 1], NPH).astype(jnp.int32)
    bnds = jnp.stack([
        jnp.stack([jnp.zeros((16,), jnp.int32),
                   jnp.broadcast_to(n0, (16,))]),
        jnp.stack([jnp.broadcast_to(n1, (16,)),
                   jnp.full((16,), NB, jnp.int32)]),
    ]).astype(jnp.int32)
    z2 = jnp.zeros((RPS, D), jnp.float32)
    z1 = jnp.zeros((RPS,), jnp.float32)
    sum_, deg = _sc_propagate(x, row2, col2, valb, bnds, z2, z1)
    out = _finalize(sum_, deg.reshape(NP, 1))
    return out[:N]


# speedup vs baseline: 1.4043x; 1.0002x over previous
import jax
import jax.numpy as jnp
from jax import lax
from jax.experimental import pallas as pl
from jax.experimental.pallas import tpu as pltpu
from jax.experimental.pallas import tpu_sc as plsc

N = 10000
D = 128
E = 320000

NC = 2
NS = 16
B = 1024
NB = 320
EP = NB * B
BPW = NB // NS
NPH = 5120
NP = NC * NPH
DUMMY = NPH
ACCR = NPH + 8
RPS = NPH // NS

import numpy as _np
_OH = [_np.eye(16, dtype=_np.float32)[j] for j in range(16)]


def _sc_body(x_hbm, row_hbm, col_hbm, valb_hbm, bnds_hbm, z2_hbm, z1_hbm,
             sum_hbm, deg_hbm,
             row_v, loc_v, col_v, valb_v, gx, ones1, bnd_v, db, acc, deg1,
             sem, isem, ssem):
    c = lax.axis_index("c")
    s = lax.axis_index("s")
    lo = c * NPH
    hi = lo + NPH

    def _fill1(r, carry):
        ones1[pl.ds(r * 16, 16)] = jnp.ones((16,), jnp.float32)
        return carry
    lax.fori_loop(0, 128 // 16, _fill1, 0)

    def _zrow(r, carry):
        for j in range(D // 16):
            gx[r, pl.ds(j * 16, 16)] = jnp.zeros((16,), jnp.float32)
        return carry
    lax.fori_loop(0, RPS + 8, _zrow, 0)

    def _zdg(r, carry):
        db[pl.ds(r * 16, 16)] = jnp.zeros((16,), jnp.float32)
        return carry
    lax.fori_loop(0, RPS // 16, _zdg, 0)

    pltpu.sync_copy(gx.at[pl.ds(0, RPS)], acc.at[pl.ds(s * RPS, RPS)])

    @pl.when(s == 0)
    def _():
        pltpu.sync_copy(gx.at[pl.ds(0, 8)], acc.at[pl.ds(NPH, 8)])
        pltpu.sync_copy(db.at[pl.ds(0, 8)], deg1.at[pl.ds(NPH, 8)])
    pltpu.sync_copy(db, deg1.at[pl.ds(s * RPS, RPS)])

    pltpu.sync_copy(bnds_hbm.at[c], bnd_v)
    blo = jnp.max(bnd_v[0, pl.ds(0, 16)])
    bhi = jnp.max(bnd_v[1, pl.ds(0, 16)])
    plsc.subcore_barrier()

    ilo = jnp.maximum((blo - s + NS - 1) // NS, 0)
    ihi = jnp.minimum((bhi - s + NS - 1) // NS, BPW)

    def _block(i, carry):
        bid = s + i * NS
        e0 = bid * B
        rb = pl.multiple_of(e0 // 128, 8)
        ips = [pltpu.async_copy(row_hbm.at[pl.ds(rb, B // 128)], row_v, isem),
               pltpu.async_copy(col_hbm.at[pl.ds(rb, B // 128)], col_v, isem),
               pltpu.async_copy(valb_hbm.at[pl.ds(e0, B)], valb_v, isem)]
        for cp in ips:
            cp.wait()

        def _remap(q, carry2):
            for j in range(8):
                rv = row_v[q, pl.ds(j * 16, 16)]
                lr = rv - lo
                m = jnp.logical_and(rv >= lo, rv < hi)
                loc_v[q, pl.ds(j * 16, 16)] = jnp.where(
                    m, lr, jnp.full((16,), DUMMY, jnp.int32))
            return carry2
        lax.fori_loop(0, B // 128, _remap, 0)

        for h in range(2):
            cps = [pltpu.async_copy(x_hbm.at[col_v.at[h * 4 + k]],
                                    gx.at[pl.ds(k * 128, 128)], sem)
                   for k in range(4)]
            for cp in cps:
                cp.wait()

            def _scale(e, carry2):
                vb = valb_v[pl.ds((h * 512 + e) * 16, 16)]
                for j in range(D // 16):
                    g = gx[e, pl.ds(j * 16, 16)]
                    gx[e, pl.ds(j * 16, 16)] = g * vb
                return carry2
            lax.fori_loop(0, 512, _scale, 0)

            sps = []
            for k in range(4):
                sps.append(pltpu.async_copy(
                    gx.at[pl.ds(k * 128, 128)],
                    acc.at[loc_v.at[h * 4 + k]], ssem, add=True))
                sps.append(pltpu.async_copy(
                    ones1, deg1.at[loc_v.at[h * 4 + k]], ssem, add=True))
            for cp in sps:
                cp.wait()
        return carry
    lax.fori_loop(ilo, ihi, _block, 0)

    plsc.subcore_barrier()

    r0 = s * RPS
    pltpu.sync_copy(acc.at[pl.ds(r0, RPS)], gx.at[pl.ds(0, RPS)])
    pltpu.sync_copy(gx.at[pl.ds(0, RPS)],
                    sum_hbm.at[pl.ds(c * NPH + r0, RPS)])
    pltpu.sync_copy(deg1.at[pl.ds(r0, RPS)], db)
    pltpu.sync_copy(db, deg_hbm.at[pl.ds(c * NPH + r0, RPS)])


@jax.jit
def _sc_propagate(x, row2, col2, valb, bnds, z2, z1):
    mesh = plsc.VectorSubcoreMesh(core_axis_name="c", subcore_axis_name="s",
                                  num_cores=NC, num_subcores=NS)
    return pl.kernel(
        _sc_body,
        out_type=(jax.ShapeDtypeStruct((NP, D), jnp.float32),
                  jax.ShapeDtypeStruct((NP,), jnp.float32)),
        mesh=mesh,
        scratch_types=(
            pltpu.VMEM((B // 128, 128), jnp.int32),
            pltpu.VMEM((B // 128, 128), jnp.int32),
            pltpu.VMEM((B // 128, 128), jnp.int32),
            pltpu.VMEM((B,), jnp.float32),
            pltpu.VMEM((512, D), jnp.float32),
            pltpu.VMEM((128,), jnp.float32),
            pltpu.VMEM((2, 16), jnp.int32),
            pltpu.VMEM((RPS,), jnp.float32),
            pltpu.VMEM_SHARED((ACCR, D), jnp.float32),
            pltpu.VMEM_SHARED((ACCR,), jnp.float32),
            pltpu.SemaphoreType.DMA,
            pltpu.SemaphoreType.DMA,
            pltpu.SemaphoreType.DMA,
        ),
        compiler_params=pltpu.CompilerParams(use_tc_tiling_on_sc=False,
                                             needs_layout_passes=False),
    )(x, row2, col2, valb, bnds, z2, z1)


def _fin_body(p_ref, d_ref, o_ref):
    dg = jnp.maximum(d_ref[...], 1.0)
    o_ref[...] = p_ref[...] / dg


@jax.jit
def _finalize(sum_, deg2d):
    R = 1024
    return pl.pallas_call(
        _fin_body,
        grid=(NP // R,),
        in_specs=[
            pl.BlockSpec((R, D), lambda i: (i, 0)),
            pl.BlockSpec((R, 1), lambda i: (i, 0)),
        ],
        out_specs=pl.BlockSpec((R, D), lambda i: (i, 0)),
        out_shape=jax.ShapeDtypeStruct((NP, D), jnp.float32),
    )(sum_, deg2d)


def kernel(x, edge_index, edge_values):
    row = edge_index[0].astype(jnp.int32)
    col = edge_index[1].astype(jnp.int32)
    val = edge_values.astype(jnp.float32)
    pad = EP - E
    row = jnp.concatenate([row, jnp.full((pad,), NP - 8, jnp.int32)])
    col = jnp.concatenate([col, jnp.zeros((pad,), jnp.int32)])
    val = jnp.concatenate([val, jnp.zeros((pad,), jnp.float32)])
    row2 = row.reshape(EP // 128, 128)
    col2 = col.reshape(EP // 128, 128)
    valb = val
    rowB = row.reshape(NB, B)
    n0 = jnp.searchsorted(rowB[:, 0], NPH).astype(jnp.int32)
    n1 = jnp.searchsorted(rowB[:, B - 1], NPH).astype(jnp.int32)
    bnds = jnp.stack([
        jnp.stack([jnp.zeros((16,), jnp.int32),
                   jnp.broadcast_to(n0, (16,))]),
        jnp.stack([jnp.broadcast_to(n1, (16,)),
                   jnp.full((16,), NB, jnp.int32)]),
    ]).astype(jnp.int32)
    z2 = jnp.zeros((RPS, D), jnp.float32)
    z1 = jnp.zeros((RPS,), jnp.float32)
    sum_, deg = _sc_propagate(x, row2, col2, valb, bnds, z2, z1)
    out = _finalize(sum_, deg.reshape(NP, 1))
    return out[:N]
